# baseline (speedup 1.0000x reference)
import functools

import jax
import jax.numpy as jnp
from jax import lax
from jax.experimental import pallas as pl
from jax.experimental.pallas import tpu as pltpu
from jax.experimental.pallas import tpu_sc as plsc

NC, NS, L = 2, 16, 16
NW = NC * NS
B, N, C1 = 16, 512, 81
BH = B // 2
NV = N // L
UPT = 5

_mesh = plsc.VectorSubcoreMesh(core_axis_name="c", subcore_axis_name="s")

_plane = pltpu.VMEM((BH, N), jnp.float32)


@functools.partial(
    pl.kernel,
    out_type=jax.ShapeDtypeStruct((NW, 2, L), jnp.float32),
    mesh=_mesh,
    compiler_params=pltpu.CompilerParams(
        use_tc_tiling_on_sc=True,
        needs_layout_passes=False,
        disable_bounds_checks=True,
    ),
    scratch_types=[_plane] * 18 + [
        pltpu.VMEM((2, L), jnp.float32),
        pltpu.SemaphoreType.DMA,
        pltpu.SemaphoreType.DMA,
    ],
)
def _sc_loss(o_hbm, t_hbm, l_hbm, out_hbm, *refs):
    (o00, o01, o02, o03, o10, o11, o12, o13,
     t00, t01, t02, t03, t10, t11, t12, t13,
     lb0, lb1, stage, sem0, sem1) = refs
    obufs = ((o00, o01, o02, o03), (o10, o11, o12, o13))
    tbufs = ((t00, t01, t02, t03), (t10, t11, t12, t13))
    lbufs = (lb0, lb1)
    sems = (sem0, sem1)

    wid = lax.axis_index("s") * NC + lax.axis_index("c")
    u0 = wid * UPT

    def start(ui):
        slot = ui % 2
        u = u0 + ui
        g = 1 + u // 2
        b0 = (u % 2) * BH
        ds = []
        for j in range(4):
            ds.append(pltpu.async_copy(
                o_hbm.at[4 * g + j, pl.ds(b0, BH), :], obufs[slot][j], sems[slot]))
            ds.append(pltpu.async_copy(
                t_hbm.at[4 * g + j, pl.ds(b0, BH), :], tbufs[slot][j], sems[slot]))
        ds.append(pltpu.async_copy(
            l_hbm.at[g, pl.ds(b0, BH), :], lbufs[slot], sems[slot]))
        return ds

    num_acc = jnp.zeros((L,), jnp.float32)
    den_acc = jnp.zeros((L,), jnp.float32)

    descs = start(0)
    for ui in range(UPT):
        slot = ui % 2
        if ui + 1 < UPT:
            nxt = start(ui + 1)
        for d in descs:
            d.wait()

        obs, tbs, lb = obufs[slot], tbufs[slot], lbufs[slot]

        def b_body(i, carry, _obs=obs, _tbs=tbs, _lb=lb):
            num, den = carry
            bi = i >> 2
            c0 = (i & 3) * 128
            for vl in range(8):
                lab = _lb[bi, pl.ds(c0 + vl * L, L)]
                m = lab == 1.0
                den = den + lab
                gacc = None
                for j in range(4):
                    o = _obs[j][bi, pl.ds(c0 + vl * L, L)]
                    t = _tbs[j][bi, pl.ds(c0 + vl * L, L)]
                    d = o - t
                    ad = jnp.abs(d)
                    mn = jnp.minimum(ad, 1.0)
                    f = mn * (ad - 0.5 * mn)
                    gacc = f if gacc is None else gacc + f
                num = num + jnp.where(m, gacc, 0.0)
            return num, den

        num_acc, den_acc = lax.fori_loop(0, 4 * BH, b_body, (num_acc, den_acc))
        if ui + 1 < UPT:
            descs = nxt

    stage[0] = num_acc
    stage[1] = den_acc
    pltpu.sync_copy(stage, out_hbm.at[wid])


def kernel(output, target, labels_target):
    o = output.transpose(2, 0, 1)
    t = target.transpose(2, 0, 1)
    lt = labels_target.transpose(2, 0, 1)
    part = _sc_loss(o, t, lt)
    s = jnp.sum(part, axis=(0, 2))
    return s[0] / (s[1] + jnp.float32(0.0001 * B * N * (C1 - 1)))


# speedup vs baseline: 1.0041x; 1.0041x over previous
import functools

import jax
import jax.numpy as jnp
from jax import lax
from jax.experimental import pallas as pl
from jax.experimental.pallas import tpu as pltpu
from jax.experimental.pallas import tpu_sc as plsc

NC, NS, L = 2, 16, 16
NW = NC * NS
B, N, C1 = 16, 512, 81
BH = B // 2
NV = N // L
UPT = 5

_mesh = plsc.VectorSubcoreMesh(core_axis_name="c", subcore_axis_name="s")

_plane = pltpu.VMEM((4 * BH, 128), jnp.float32)


@functools.partial(
    pl.kernel,
    out_type=jax.ShapeDtypeStruct((NW, 2, L), jnp.float32),
    mesh=_mesh,
    compiler_params=pltpu.CompilerParams(
        use_tc_tiling_on_sc=True,
        needs_layout_passes=False,
        disable_bounds_checks=True,
    ),
    scratch_types=[_plane] * 18 + [
        pltpu.VMEM((2, L), jnp.float32),
        pltpu.SemaphoreType.DMA,
        pltpu.SemaphoreType.DMA,
    ],
)
def _sc_loss(o_hbm, t_hbm, l_hbm, out_hbm, *refs):
    (o00, o01, o02, o03, o10, o11, o12, o13,
     t00, t01, t02, t03, t10, t11, t12, t13,
     lb0, lb1, stage, sem0, sem1) = refs
    obufs = ((o00, o01, o02, o03), (o10, o11, o12, o13))
    tbufs = ((t00, t01, t02, t03), (t10, t11, t12, t13))
    lbufs = (lb0, lb1)
    sems = (sem0, sem1)

    wid = lax.axis_index("s") * NC + lax.axis_index("c")
    u0 = wid * UPT

    def start(ui):
        slot = ui % 2
        u = u0 + ui
        g = 1 + u // 2
        b0 = u % 2
        ds = []
        for j in range(4):
            ds.append(pltpu.async_copy(
                o_hbm.at[4 * g + j, b0], obufs[slot][j], sems[slot]))
            ds.append(pltpu.async_copy(
                t_hbm.at[4 * g + j, b0], tbufs[slot][j], sems[slot]))
        ds.append(pltpu.async_copy(
            l_hbm.at[g, b0], lbufs[slot], sems[slot]))
        return ds

    num_acc = jnp.zeros((L,), jnp.float32)
    den_acc = jnp.zeros((L,), jnp.float32)

    descs = start(0)
    for ui in range(UPT):
        slot = ui % 2
        if ui + 1 < UPT:
            nxt = start(ui + 1)
        for d in descs:
            d.wait()

        obs, tbs, lb = obufs[slot], tbufs[slot], lbufs[slot]

        def b_body(r, carry, _obs=obs, _tbs=tbs, _lb=lb):
            num, den = carry
            for vl in range(8):
                lab = _lb[r, pl.ds(vl * L, L)]
                m = lab == 1.0
                den = den + lab
                gacc = None
                for j in range(4):
                    o = _obs[j][r, pl.ds(vl * L, L)]
                    t = _tbs[j][r, pl.ds(vl * L, L)]
                    d = o - t
                    ad = jnp.abs(d)
                    mn = jnp.minimum(ad, 1.0)
                    f = mn * (ad - 0.5 * mn)
                    gacc = f if gacc is None else gacc + f
                num = num + jnp.where(m, gacc, 0.0)
            return num, den

        num_acc, den_acc = lax.fori_loop(0, 4 * BH, b_body, (num_acc, den_acc))
        if ui + 1 < UPT:
            descs = nxt

    stage[0] = num_acc
    stage[1] = den_acc
    pltpu.sync_copy(stage, out_hbm.at[wid])


def _tileview(x, c):
    v = x.transpose(2, 0, 1).reshape(c, 2, 8, 4, 128)
    return v.transpose(0, 1, 3, 2, 4).reshape(c, 2, 32, 128)


def kernel(output, target, labels_target):
    o = _tileview(output, 4 * C1)
    t = _tileview(target, 4 * C1)
    lt = _tileview(labels_target, C1)
    part = _sc_loss(o, t, lt)
    s = jnp.sum(part, axis=(0, 2))
    return s[0] / (s[1] + jnp.float32(0.0001 * B * N * (C1 - 1)))


# speedup vs baseline: 1.0586x; 1.0543x over previous
import functools

import jax
import jax.numpy as jnp
from jax import lax
from jax.experimental import pallas as pl
from jax.experimental.pallas import tpu as pltpu
from jax.experimental.pallas import tpu_sc as plsc

NC, NS, L = 2, 16, 16
NW = NC * NS
B, N, C1 = 16, 512, 81
BH = B // 2
NV = N // L
GSPLIT = 49
UPT = 2 * (GSPLIT - 1) // NW
NTC = C1 - GSPLIT

_mesh = plsc.VectorSubcoreMesh(core_axis_name="c", subcore_axis_name="s")

_plane = pltpu.VMEM((BH, N), jnp.float32)


@functools.partial(
    pl.kernel,
    out_type=jax.ShapeDtypeStruct((NW, 2, L), jnp.float32),
    mesh=_mesh,
    compiler_params=pltpu.CompilerParams(
        use_tc_tiling_on_sc=True,
        needs_layout_passes=False,
        disable_bounds_checks=True,
    ),
    scratch_types=[_plane] * 18 + [
        pltpu.VMEM((2, L), jnp.float32),
        pltpu.SemaphoreType.DMA,
        pltpu.SemaphoreType.DMA,
    ],
)
def _sc_loss(o_hbm, t_hbm, l_hbm, out_hbm, *refs):
    (o00, o01, o02, o03, o10, o11, o12, o13,
     t00, t01, t02, t03, t10, t11, t12, t13,
     lb0, lb1, stage, sem0, sem1) = refs
    obufs = ((o00, o01, o02, o03), (o10, o11, o12, o13))
    tbufs = ((t00, t01, t02, t03), (t10, t11, t12, t13))
    lbufs = (lb0, lb1)
    sems = (sem0, sem1)

    wid = lax.axis_index("s") * NC + lax.axis_index("c")
    u0 = wid * UPT

    def start(ui):
        slot = ui % 2
        u = u0 + ui
        g = 1 + u // 2
        b0 = (u % 2) * BH
        ds = []
        for j in range(4):
            ds.append(pltpu.async_copy(
                o_hbm.at[4 * g + j, pl.ds(b0, BH), :], obufs[slot][j], sems[slot]))
            ds.append(pltpu.async_copy(
                t_hbm.at[4 * g + j, pl.ds(b0, BH), :], tbufs[slot][j], sems[slot]))
        ds.append(pltpu.async_copy(
            l_hbm.at[g, pl.ds(b0, BH), :], lbufs[slot], sems[slot]))
        return ds

    num_acc = jnp.zeros((L,), jnp.float32)
    den_acc = jnp.zeros((L,), jnp.float32)

    descs = start(0)
    for ui in range(UPT):
        slot = ui % 2
        if ui + 1 < UPT:
            nxt = start(ui + 1)
        for d in descs:
            d.wait()

        obs, tbs, lb = obufs[slot], tbufs[slot], lbufs[slot]

        def b_body(bi, carry, _obs=obs, _tbs=tbs, _lb=lb):
            num, den = carry
            for v in range(NV):
                lab = _lb[bi, pl.ds(v * L, L)]
                m = lab == 1.0
                den = den + lab
                gacc = None
                for j in range(4):
                    o = _obs[j][bi, pl.ds(v * L, L)]
                    t = _tbs[j][bi, pl.ds(v * L, L)]
                    d = o - t
                    ad = jnp.abs(d)
                    mn = jnp.minimum(ad, 1.0)
                    f = mn * (ad - 0.5 * mn)
                    gacc = f if gacc is None else gacc + f
                num = num + jnp.where(m, gacc, 0.0)
            return num, den

        num_acc, den_acc = lax.fori_loop(0, BH, b_body, (num_acc, den_acc))
        if ui + 1 < UPT:
            descs = nxt

    stage[0] = num_acc
    stage[1] = den_acc
    pltpu.sync_copy(stage, out_hbm.at[wid])


def _tc_body(o_ref, t_ref, l_ref, out_ref):
    i = pl.program_id(0)
    lab = l_ref[0]
    m = lab == 1.0
    d = o_ref[...] - t_ref[...]
    ad = jnp.abs(d)
    mn = jnp.minimum(ad, 1.0)
    f = mn * (ad - 0.5 * mn)
    num = jnp.sum(jnp.where(m[None], f, 0.0))
    den = jnp.sum(lab)

    vals = jnp.stack([num, den]).reshape(1, 2)

    @pl.when(i == 0)
    def _():
        out_ref[...] = vals

    @pl.when(i > 0)
    def _():
        out_ref[...] += vals


_tc_loss = pl.pallas_call(
    _tc_body,
    grid=(NTC,),
    in_specs=[
        pl.BlockSpec((4, B, N), lambda i: (GSPLIT + i, 0, 0)),
        pl.BlockSpec((4, B, N), lambda i: (GSPLIT + i, 0, 0)),
        pl.BlockSpec((1, B, N), lambda i: (GSPLIT + i, 0, 0)),
    ],
    out_specs=pl.BlockSpec((1, 2), lambda i: (0, 0)),
    out_shape=jax.ShapeDtypeStruct((1, 2), jnp.float32),
)


def kernel(output, target, labels_target):
    o = output.transpose(2, 0, 1)
    t = target.transpose(2, 0, 1)
    lt = labels_target.transpose(2, 0, 1)
    part_sc = _sc_loss(o, t, lt)
    part_tc = _tc_loss(o, t, lt)
    s = jnp.sum(part_sc, axis=(0, 2)) + part_tc[0]
    return s[0] / (s[1] + jnp.float32(0.0001 * B * N * (C1 - 1)))


# speedup vs baseline: 1.2069x; 1.1401x over previous
import functools

import jax
import jax.numpy as jnp
from jax import lax
from jax.experimental import pallas as pl
from jax.experimental.pallas import tpu as pltpu
from jax.experimental.pallas import tpu_sc as plsc

NC, NS, L = 2, 16, 16
NW = NC * NS
B, N, C1 = 16, 512, 81
BH = B // 2
NV = N // L
UPT = 3
TCB = 12

_mesh = plsc.VectorSubcoreMesh(core_axis_name="c", subcore_axis_name="s")

_plane = pltpu.VMEM((BH, N), jnp.float32)


@functools.partial(
    pl.kernel,
    out_type=jax.ShapeDtypeStruct((NW, 2, L), jnp.float32),
    mesh=_mesh,
    compiler_params=pltpu.CompilerParams(
        use_tc_tiling_on_sc=True,
        needs_layout_passes=False,
        disable_bounds_checks=True,
    ),
    scratch_types=[_plane] * 18 + [
        pltpu.VMEM((2, L), jnp.float32),
        pltpu.SemaphoreType.DMA,
        pltpu.SemaphoreType.DMA,
    ],
)
def _sc_loss(o_hbm, t_hbm, l_hbm, out_hbm, *refs):
    (o00, o01, o02, o03, o10, o11, o12, o13,
     t00, t01, t02, t03, t10, t11, t12, t13,
     lb0, lb1, stage, sem0, sem1) = refs
    obufs = ((o00, o01, o02, o03), (o10, o11, o12, o13))
    tbufs = ((t00, t01, t02, t03), (t10, t11, t12, t13))
    lbufs = (lb0, lb1)
    sems = (sem0, sem1)

    wid = lax.axis_index("s") * NC + lax.axis_index("c")
    u0 = wid * UPT

    def start(ui):
        slot = ui % 2
        u = u0 + ui
        q = u // 2
        g = jnp.where(q == 47, 80, q + 1)
        b0 = (u % 2) * BH
        ds = []
        for j in range(4):
            ds.append(pltpu.async_copy(
                o_hbm.at[4 * g + j, pl.ds(b0, BH), :], obufs[slot][j], sems[slot]))
            ds.append(pltpu.async_copy(
                t_hbm.at[4 * g + j, pl.ds(b0, BH), :], tbufs[slot][j], sems[slot]))
        ds.append(pltpu.async_copy(
            l_hbm.at[g, pl.ds(b0, BH), :], lbufs[slot], sems[slot]))
        return ds

    num_acc = jnp.zeros((L,), jnp.float32)
    den_acc = jnp.zeros((L,), jnp.float32)

    descs = start(0)
    for ui in range(UPT):
        slot = ui % 2
        if ui + 1 < UPT:
            nxt = start(ui + 1)
        for d in descs:
            d.wait()

        obs, tbs, lb = obufs[slot], tbufs[slot], lbufs[slot]

        def b_body(bi, carry, _obs=obs, _tbs=tbs, _lb=lb):
            num, den = carry
            for v in range(NV):
                lab = _lb[bi, pl.ds(v * L, L)]
                m = lab == 1.0
                den = den + lab
                gacc = None
                for j in range(4):
                    o = _obs[j][bi, pl.ds(v * L, L)]
                    t = _tbs[j][bi, pl.ds(v * L, L)]
                    d = o - t
                    ad = jnp.abs(d)
                    mn = jnp.minimum(ad, 1.0)
                    f = mn * (ad - 0.5 * mn)
                    gacc = f if gacc is None else gacc + f
                num = num + jnp.where(m, gacc, 0.0)
            return num, den

        num_acc, den_acc = lax.fori_loop(0, BH, b_body, (num_acc, den_acc))
        if ui + 1 < UPT:
            descs = nxt

    stage[0] = num_acc
    stage[1] = den_acc
    pltpu.sync_copy(stage, out_hbm.at[wid])


def _tc_body(o_ref, t_ref, l_ref, out_ref):
    i = pl.program_id(0)
    lab = l_ref[...]
    m = lab == 1.0
    d = o_ref[...] - t_ref[...]
    ad = jnp.abs(d)
    mn = jnp.minimum(ad, 1.0)
    f = mn * (ad - 0.5 * mn)
    fm = jnp.where(m[:, None].repeat(4, 1).reshape(16, B, N), f, 0.0)
    num = jnp.sum(fm)
    den = jnp.sum(lab)

    vals = jnp.stack([num, den]).reshape(1, 2)

    @pl.when(i == 0)
    def _():
        out_ref[...] = vals

    @pl.when(i > 0)
    def _():
        out_ref[...] += vals


_tc_loss = pl.pallas_call(
    _tc_body,
    grid=(8,),
    in_specs=[
        pl.BlockSpec((16, B, N), lambda i: (TCB + i, 0, 0)),
        pl.BlockSpec((16, B, N), lambda i: (TCB + i, 0, 0)),
        pl.BlockSpec((4, B, N), lambda i: (TCB + i, 0, 0)),
    ],
    out_specs=pl.BlockSpec((1, 2), lambda i: (0, 0)),
    out_shape=jax.ShapeDtypeStruct((1, 2), jnp.float32),
)


def kernel(output, target, labels_target):
    o = output.transpose(2, 0, 1)
    t = target.transpose(2, 0, 1)
    lt = labels_target.transpose(2, 0, 1)
    part_sc = _sc_loss(o, t, lt)
    part_tc = _tc_loss(o, t, lt)
    s = jnp.sum(part_sc, axis=(0, 2)) + part_tc[0]
    return s[0] / (s[1] + jnp.float32(0.0001 * B * N * (C1 - 1)))


# speedup vs baseline: 1.2794x; 1.0601x over previous
import functools

import jax
import jax.numpy as jnp
from jax import lax
from jax.experimental import pallas as pl
from jax.experimental.pallas import tpu as pltpu
from jax.experimental.pallas import tpu_sc as plsc

NC, NS, L = 2, 16, 16
NW = NC * NS
B, N, C1 = 16, 512, 81
BH = B // 2
NV = N // L
UPT = 2
TCB = 8

_mesh = plsc.VectorSubcoreMesh(core_axis_name="c", subcore_axis_name="s")

_plane = pltpu.VMEM((BH, N), jnp.float32)


@functools.partial(
    pl.kernel,
    out_type=jax.ShapeDtypeStruct((NW, 2, L), jnp.float32),
    mesh=_mesh,
    compiler_params=pltpu.CompilerParams(
        use_tc_tiling_on_sc=True,
        needs_layout_passes=False,
        disable_bounds_checks=True,
    ),
    scratch_types=[_plane] * 18 + [
        pltpu.VMEM((2, L), jnp.float32),
        pltpu.SemaphoreType.DMA,
        pltpu.SemaphoreType.DMA,
    ],
)
def _sc_loss(o_hbm, t_hbm, l_hbm, out_hbm, *refs):
    (o00, o01, o02, o03, o10, o11, o12, o13,
     t00, t01, t02, t03, t10, t11, t12, t13,
     lb0, lb1, stage, sem0, sem1) = refs
    obufs = ((o00, o01, o02, o03), (o10, o11, o12, o13))
    tbufs = ((t00, t01, t02, t03), (t10, t11, t12, t13))
    lbufs = (lb0, lb1)
    sems = (sem0, sem1)

    wid = lax.axis_index("s") * NC + lax.axis_index("c")
    u0 = wid * UPT

    def start(ui):
        slot = ui % 2
        u = u0 + ui
        q = u // 2
        g = jnp.where(q == 31, 80, q + 1)
        b0 = (u % 2) * BH
        ds = []
        for j in range(4):
            ds.append(pltpu.async_copy(
                o_hbm.at[4 * g + j, pl.ds(b0, BH), :], obufs[slot][j], sems[slot]))
            ds.append(pltpu.async_copy(
                t_hbm.at[4 * g + j, pl.ds(b0, BH), :], tbufs[slot][j], sems[slot]))
        ds.append(pltpu.async_copy(
            l_hbm.at[g, pl.ds(b0, BH), :], lbufs[slot], sems[slot]))
        return ds

    num_acc = jnp.zeros((L,), jnp.float32)
    den_acc = jnp.zeros((L,), jnp.float32)

    descs = start(0)
    for ui in range(UPT):
        slot = ui % 2
        if ui + 1 < UPT:
            nxt = start(ui + 1)
        for d in descs:
            d.wait()

        obs, tbs, lb = obufs[slot], tbufs[slot], lbufs[slot]

        def b_body(bi, carry, _obs=obs, _tbs=tbs, _lb=lb):
            num, den = carry
            for v in range(NV):
                lab = _lb[bi, pl.ds(v * L, L)]
                m = lab == 1.0
                den = den + lab
                gacc = None
                for j in range(4):
                    o = _obs[j][bi, pl.ds(v * L, L)]
                    t = _tbs[j][bi, pl.ds(v * L, L)]
                    d = o - t
                    ad = jnp.abs(d)
                    mn = jnp.minimum(ad, 1.0)
                    f = mn * (ad - 0.5 * mn)
                    gacc = f if gacc is None else gacc + f
                num = num + jnp.where(m, gacc, 0.0)
            return num, den

        num_acc, den_acc = lax.fori_loop(0, BH, b_body, (num_acc, den_acc))
        if ui + 1 < UPT:
            descs = nxt

    stage[0] = num_acc
    stage[1] = den_acc
    pltpu.sync_copy(stage, out_hbm.at[wid])


def _tc_body(o_ref, t_ref, l_ref, out_ref):
    i = pl.program_id(0)
    lab = l_ref[...]
    m = lab == 1.0
    d = o_ref[...] - t_ref[...]
    ad = jnp.abs(d)
    mn = jnp.minimum(ad, 1.0)
    f = mn * (ad - 0.5 * mn)
    fm = jnp.where(m[:, None].repeat(4, 1).reshape(16, B, N), f, 0.0)
    num = jnp.sum(fm)
    den = jnp.sum(lab)

    vals = jnp.stack([num, den]).reshape(1, 2)

    @pl.when(i == 0)
    def _():
        out_ref[...] = vals

    @pl.when(i > 0)
    def _():
        out_ref[...] += vals


_tc_loss = pl.pallas_call(
    _tc_body,
    grid=(12,),
    in_specs=[
        pl.BlockSpec((16, B, N), lambda i: (TCB + i, 0, 0)),
        pl.BlockSpec((16, B, N), lambda i: (TCB + i, 0, 0)),
        pl.BlockSpec((4, B, N), lambda i: (TCB + i, 0, 0)),
    ],
    out_specs=pl.BlockSpec((1, 2), lambda i: (0, 0)),
    out_shape=jax.ShapeDtypeStruct((1, 2), jnp.float32),
)


def kernel(output, target, labels_target):
    o = output.transpose(2, 0, 1)
    t = target.transpose(2, 0, 1)
    lt = labels_target.transpose(2, 0, 1)
    part_sc = _sc_loss(o, t, lt)
    part_tc = _tc_loss(o, t, lt)
    s = jnp.sum(part_sc, axis=(0, 2)) + part_tc[0]
    return s[0] / (s[1] + jnp.float32(0.0001 * B * N * (C1 - 1)))
